# baseline (speedup 1.0000x reference)
import functools

import jax
import jax.numpy as jnp
from jax.experimental import pallas as pl
from jax.experimental.pallas import tpu as pltpu

_BS = 4
_NQ = 1000
_NC = 91
_PRE_TOPK = 10000
_KEEP = 100
_IOU_THR = 0.7
_BIG = 1 << 30
_KS, _KBX1, _KBY1, _KBX2, _KBY2, _KX1O, _KY1O, _KX2O, _KY2O, _KAREA = range(10)


def _nms_body(logits_ref, boxes_t_ref, boxes_raw_ref, scale_ref,
              s_out_ref, l_out_ref, x1_out_ref, y1_out_ref, x2_out_ref,
              y2_out_ref, msb0_ref, msb1_ref, msb2_ref, msb3_ref, s_ref,
              rmax_ref, rbest_ref, kf_ref, kl_ref):
    msb_refs = [msb0_ref, msb1_ref, msb2_ref, msb3_ref]
    row_iota = jax.lax.broadcasted_iota(jnp.int32, (_NC, _NQ), 0)
    col_iota = jax.lax.broadcasted_iota(jnp.int32, (_NC, _NQ), 1)
    fidx = col_iota * _NC + row_iota
    riota = jax.lax.broadcasted_iota(jnp.int32, (_NC, 1), 0)
    col1 = jax.lax.broadcasted_iota(jnp.int32, (1, _NQ), 1)
    lane128 = jax.lax.broadcasted_iota(jnp.int32, (1, 128), 1)

    sbits_all = []
    sw_all = []
    sh_all = []
    for b in range(_BS):
        s_b = jax.nn.sigmoid(logits_ref[b])
        s_ref[b * _NC:(b + 1) * _NC, :] = s_b
        sbits_all.append(jax.lax.bitcast_convert_type(s_b, jnp.int32))
        sw_all.append(jnp.sum(scale_ref[b:b + 1, 0:1]))
        sh_all.append(jnp.sum(scale_ref[b:b + 1, 1:2]))

    kf_ref[...] = jnp.zeros((10 * _BS, 128), jnp.float32)
    kl_ref[...] = jnp.zeros((_BS, 128), jnp.int32)

    def bs_val(_, state):
        out = []
        for b in range(_BS):
            lo, hi = state[2 * b], state[2 * b + 1]
            mid = (lo + hi) // 2
            take_hi = jnp.sum((sbits_all[b] > mid).astype(jnp.int32)) >= _PRE_TOPK
            out.append(jnp.where(take_hi, mid, lo))
            out.append(jnp.where(take_hi, hi, mid))
        return tuple(out)

    st0 = (jnp.int32(-1), jnp.int32(0x3F800000)) * _BS
    st = jax.lax.fori_loop(0, 31, bs_val, st0)
    taus = [st[2 * b + 1] for b in range(_BS)]
    needs = [
        _PRE_TOPK - jnp.sum((sbits_all[b] > taus[b]).astype(jnp.int32))
        for b in range(_BS)
    ]
    ties = [sbits_all[b] == taus[b] for b in range(_BS)]

    def bs_idx(_, state):
        out = []
        for b in range(_BS):
            lo, hi = state[2 * b], state[2 * b + 1]
            mid = (lo + hi) // 2
            cnt = jnp.sum((ties[b] & (fidx < mid)).astype(jnp.int32))
            take_hi = cnt >= needs[b]
            out.append(jnp.where(take_hi, lo, mid))
            out.append(jnp.where(take_hi, mid, hi))
        return tuple(out)

    st0 = (jnp.int32(0), jnp.int32(_NQ * _NC)) * _BS
    st = jax.lax.fori_loop(0, 17, bs_idx, st0)
    mstars = [st[2 * b + 1] for b in range(_BS)]

    off_units = []
    for b in range(_BS):
        elig = (sbits_all[b] > taus[b]) | (ties[b] & (fidx < mstars[b]))
        msb0 = jnp.where(elig, sbits_all[b], jnp.int32(-1))
        msb_refs[b][...] = msb0
        rmax0 = jnp.max(msb0, axis=1, keepdims=True)
        qmin0 = jnp.min(jnp.where(msb0 == rmax0, col_iota, _BIG),
                        axis=1, keepdims=True)
        rmax_ref[:, b:b + 1] = rmax0
        rbest_ref[:, b:b + 1] = qmin0 * _NC + riota
        cxt = boxes_t_ref[b, 0:1, :]
        cyt = boxes_t_ref[b, 1:2, :]
        wt = boxes_t_ref[b, 2:3, :]
        ht = boxes_t_ref[b, 3:4, :]
        x1t = (cxt - 0.5 * wt) * sw_all[b]
        y1t = (cyt - 0.5 * ht) * sh_all[b]
        x2t = (cxt + 0.5 * wt) * sw_all[b]
        y2t = (cyt + 0.5 * ht) * sh_all[b]
        qmax = jnp.maximum(jnp.maximum(x1t, x2t), jnp.maximum(y1t, y2t))
        row_any = jnp.any(elig, axis=0, keepdims=True)
        maxc = jnp.max(jnp.where(row_any, qmax, jnp.float32(-3.4e38)))
        off_units.append(maxc + 1.0)

    halfsign = jnp.concatenate(
        [jnp.full((1, 2), -0.5, jnp.float32), jnp.full((1, 2), 0.5, jnp.float32)],
        axis=1)

    def cond(carry):
        live = None
        for b in range(_BS):
            lb = (carry[2 * b] < _KEEP) & (carry[2 * b + 1] == 0)
            live = lb if live is None else (live | lb)
        return live

    def body(carry):
        rm = rmax_ref[...]
        mb = jnp.max(rm, axis=0, keepdims=True)
        fpick = jnp.min(jnp.where(rm == mb, rbest_ref[...], _BIG),
                        axis=0, keepdims=True)
        out = []
        for b in range(_BS):
            cnt_b, done_b = carry[2 * b], carry[2 * b + 1]
            f_b = jnp.sum(fpick[:, b:b + 1])
            mb_b = jnp.sum(mb[:, b:b + 1])
            live_b = (cnt_b < _KEEP) & (done_b == 0) & (mb_b >= 0)
            newdone_b = jnp.where(mb_b < 0, jnp.int32(1), done_b)
            q = f_b // _NC
            c = f_b % _NC
            r = b * _NC + c
            row = msb_refs[b][pl.ds(c, 1), :]
            row = jnp.where((col1 == q) & live_b, jnp.int32(-1), row)
            msb_refs[b][pl.ds(c, 1), :] = row
            nrmax = jnp.max(row, axis=1, keepdims=True)
            nqmin = jnp.min(jnp.where(row == nrmax, col1, _BIG),
                            axis=1, keepdims=True)
            rmax_ref[pl.ds(c, 1), b:b + 1] = nrmax
            rbest_ref[pl.ds(c, 1), b:b + 1] = nqmin * _NC + c
            srow = s_ref[pl.ds(r, 1), :]
            sval = jnp.sum(jnp.where(col1 == q, srow, 0.0),
                           axis=1, keepdims=True)
            braw = boxes_raw_ref[b, pl.ds(q, 1), :]
            cxy2 = jnp.concatenate([braw[:, 0:2], braw[:, 0:2]], axis=1)
            wh2 = jnp.concatenate([braw[:, 2:4], braw[:, 2:4]], axis=1)
            svec = scale_ref[b:b + 1, :]
            pxy = (cxy2 + halfsign * wh2) * svec
            co = pxy + c.astype(jnp.float32) * off_units[b]
            d = jnp.maximum(co[:, 2:4] - co[:, 0:2], 0.0)
            carea = d[:, 0:1] * d[:, 1:2]
            kx1o = kf_ref[10 * b + _KX1O:10 * b + _KX1O + 1, :]
            ky1o = kf_ref[10 * b + _KY1O:10 * b + _KY1O + 1, :]
            kx2o = kf_ref[10 * b + _KX2O:10 * b + _KX2O + 1, :]
            ky2o = kf_ref[10 * b + _KY2O:10 * b + _KY2O + 1, :]
            karea = kf_ref[10 * b + _KAREA:10 * b + _KAREA + 1, :]
            ix = jnp.maximum(
                jnp.minimum(kx2o, co[:, 2:3]) - jnp.maximum(kx1o, co[:, 0:1]),
                0.0)
            iy = jnp.maximum(
                jnp.minimum(ky2o, co[:, 3:4]) - jnp.maximum(ky1o, co[:, 1:2]),
                0.0)
            inter = ix * iy
            union = karea + carea - inter
            iou = inter / jnp.maximum(union, 1e-9)
            supp = jnp.max(iou) > _IOU_THR
            keep_b = jnp.logical_not(supp) & live_b
            cm = (lane128 == cnt_b) & keep_b
            upd_f = [(_KS, sval), (_KBX1, pxy[:, 0:1]), (_KBY1, pxy[:, 1:2]),
                     (_KBX2, pxy[:, 2:3]), (_KBY2, pxy[:, 3:4]),
                     (_KX1O, co[:, 0:1]), (_KY1O, co[:, 1:2]),
                     (_KX2O, co[:, 2:3]), (_KY2O, co[:, 3:4]), (_KAREA, carea)]
            for slot, val in upd_f:
                old = kf_ref[10 * b + slot:10 * b + slot + 1, :]
                kf_ref[10 * b + slot:10 * b + slot + 1, :] = (
                    jnp.where(cm, val, old))
            kl_old = kl_ref[b:b + 1, :]
            kl_ref[b:b + 1, :] = jnp.where(cm, c, kl_old)
            out.append(cnt_b + keep_b.astype(jnp.int32))
            out.append(newdone_b)
        return tuple(out)

    carry0 = (jnp.int32(0),) * (2 * _BS)
    final = jax.lax.while_loop(cond, body, carry0)

    for b in range(_BS):
        cnt_b = final[2 * b]
        padm = lane128 >= cnt_b

        def pad(vec):
            return jnp.where(padm, jnp.broadcast_to(vec[:, 0:1], vec.shape),
                             vec)

        s_out_ref[b:b + 1, :] = pad(kf_ref[10 * b + _KS:10 * b + _KS + 1, :])[:, :_KEEP]
        l_out_ref[b:b + 1, :] = pad(kl_ref[b:b + 1, :])[:, :_KEEP]
        x1_out_ref[b:b + 1, :] = pad(kf_ref[10 * b + _KBX1:10 * b + _KBX1 + 1, :])[:, :_KEEP]
        y1_out_ref[b:b + 1, :] = pad(kf_ref[10 * b + _KBY1:10 * b + _KBY1 + 1, :])[:, :_KEEP]
        x2_out_ref[b:b + 1, :] = pad(kf_ref[10 * b + _KBX2:10 * b + _KBX2 + 1, :])[:, :_KEEP]
        y2_out_ref[b:b + 1, :] = pad(kf_ref[10 * b + _KBY2:10 * b + _KBY2 + 1, :])[:, :_KEEP]


def _build_call(interpret=False):
    f32 = jnp.float32
    return pl.pallas_call(
        _nms_body,
        out_shape=[
            jax.ShapeDtypeStruct((_BS, _KEEP), f32),
            jax.ShapeDtypeStruct((_BS, _KEEP), jnp.int32),
            jax.ShapeDtypeStruct((_BS, _KEEP), f32),
            jax.ShapeDtypeStruct((_BS, _KEEP), f32),
            jax.ShapeDtypeStruct((_BS, _KEEP), f32),
            jax.ShapeDtypeStruct((_BS, _KEEP), f32),
        ],
        scratch_shapes=[
            pltpu.VMEM((_NC, _NQ), jnp.int32),
            pltpu.VMEM((_NC, _NQ), jnp.int32),
            pltpu.VMEM((_NC, _NQ), jnp.int32),
            pltpu.VMEM((_NC, _NQ), jnp.int32),
            pltpu.VMEM((_BS * _NC, _NQ), f32),
            pltpu.VMEM((_NC, _BS), jnp.int32),
            pltpu.VMEM((_NC, _BS), jnp.int32),
            pltpu.VMEM((10 * _BS, 128), f32),
            pltpu.VMEM((_BS, 128), jnp.int32),
        ],
        interpret=interpret,
    )


@jax.jit
def kernel(pred_logits, pred_boxes, target_sizes):
    logits_t = jnp.transpose(pred_logits, (0, 2, 1))
    boxes_t = jnp.transpose(pred_boxes, (0, 2, 1))
    img_h = target_sizes[:, 0].astype(jnp.float32)
    img_w = target_sizes[:, 1].astype(jnp.float32)
    scale = jnp.stack([img_w, img_h, img_w, img_h], axis=1)
    scores, labels, x1, y1, x2, y2 = _build_call()(
        logits_t, boxes_t, pred_boxes, scale)
    boxes = jnp.stack([x1, y1, x2, y2], axis=-1)
    return scores, labels, boxes
